# Initial kernel scaffold; baseline (speedup 1.0000x reference)
import jax
import jax.numpy as jnp
from jax.experimental import pallas as pl


def kernel(x, edge_index, edge_attr, batch, node_W, node_b, edge_W, edge_b, g1_ll_W, g1_ll_b, g1_lr_W, g1_lr_b, g1_le_W, g1_att, g1_bias, g2_ll_W, g2_ll_b, g2_lr_W, g2_lr_b, g2_le_W, g2_att, g2_bias, mlp_W1, mlp_b1, mlp_W2, mlp_b2):
    raise NotImplementedError("write your pallas kernel here")


# speedup vs baseline: 8.1746x; 8.1746x over previous
import functools

import jax
import jax.numpy as jnp
from jax import lax
from jax.experimental import pallas as pl
from jax.experimental.pallas import tpu as pltpu
from jax.experimental.pallas import tpu_sc as plsc

_N = 10000
_E = 320000
_H = 16
_C = 8
_D = 128
_DE = 16

_NC = 2
_NS = 16
_NW = _NC * _NS
_BLK = 16

_NPAD = 10112
_ROWS_T = _NPAD // _NS
_ES = _E + _N
_ESPAD = 330240
_EBLK = _ESPAD // (_NW * _BLK)
_LBLK = _E // (_NW * _BLK)

_f32 = jnp.float32
_i32 = jnp.int32


def _mm(a, bT, bias, act=None, bm=1024):
    M, K = a.shape
    _, Nn = bT.shape

    def kern(a_ref, b_ref, bias_ref, o_ref):
        r = jnp.dot(a_ref[...], b_ref[...], preferred_element_type=_f32)
        r = r + bias_ref[...]
        if act == "relu":
            r = jnp.maximum(r, 0.0)
        o_ref[...] = r

    return pl.pallas_call(
        kern,
        grid=(pl.cdiv(M, bm),),
        in_specs=[
            pl.BlockSpec((bm, K), lambda i: (i, 0)),
            pl.BlockSpec((K, Nn), lambda i: (0, 0)),
            pl.BlockSpec((1, Nn), lambda i: (0, 0)),
        ],
        out_specs=pl.BlockSpec((bm, Nn), lambda i: (i, 0)),
        out_shape=jax.ShapeDtypeStruct((M, Nn), _f32),
    )(a, bT, bias.reshape(1, -1))


def _loopattr_finalize(sc0, sc1, bm=1024):
    def kern(a_ref, b_ref, o_ref):
        t = a_ref[...] + b_ref[...]
        s = t[:, 0:16]
        cnt = t[:, 16:32]
        o_ref[...] = s / jnp.maximum(cnt, 1.0)

    return pl.pallas_call(
        kern,
        grid=(pl.cdiv(_NPAD, bm),),
        in_specs=[
            pl.BlockSpec((bm, 32), lambda i: (i, 0)),
            pl.BlockSpec((bm, 32), lambda i: (i, 0)),
        ],
        out_specs=pl.BlockSpec((bm, 16), lambda i: (i, 0)),
        out_shape=jax.ShapeDtypeStruct((_NPAD, 16), _f32),
    )(sc0, sc1)


def _combine(u0, u1, d0, d1, rep, bias, bm=1024):
    def kern(u0_ref, u1_ref, d0_ref, d1_ref, r_ref, b_ref, hn_ref, dt_ref):
        dent = d0_ref[...] + d1_ref[...]
        den128 = jnp.dot(dent, r_ref[...], preferred_element_type=_f32)
        out = (u0_ref[...] + u1_ref[...]) / (den128 + 1e-16)
        hn_ref[...] = jnp.maximum(out + b_ref[...], 0.0)
        dt_ref[...] = dent

    return pl.pallas_call(
        kern,
        grid=(pl.cdiv(_NPAD, bm),),
        in_specs=[
            pl.BlockSpec((bm, _D), lambda i: (i, 0)),
            pl.BlockSpec((bm, _D), lambda i: (i, 0)),
            pl.BlockSpec((bm, _H), lambda i: (i, 0)),
            pl.BlockSpec((bm, _H), lambda i: (i, 0)),
            pl.BlockSpec((_H, _D), lambda i: (0, 0)),
            pl.BlockSpec((1, _D), lambda i: (0, 0)),
        ],
        out_specs=[
            pl.BlockSpec((bm, _D), lambda i: (i, 0)),
            pl.BlockSpec((bm, _H), lambda i: (i, 0)),
        ],
        out_shape=[
            jax.ShapeDtypeStruct((_NPAD, _D), _f32),
            jax.ShapeDtypeStruct((_NPAD, _H), _f32),
        ],
    )(u0, u1, d0, d1, rep, bias.reshape(1, -1))


def _mlp(xo, h2, w1x, w1h, b1, w2row, b2, bm=1024):
    def kern(x_ref, h_ref, wx_ref, wh_ref, b1_ref, w2_ref, b2_ref, o_ref):
        z1 = jnp.dot(x_ref[...], wx_ref[...], preferred_element_type=_f32)
        z1 = z1 + jnp.dot(h_ref[...], wh_ref[...], preferred_element_type=_f32)
        z1 = jnp.maximum(z1 + b1_ref[...], 0.0)
        zs = jnp.sum(z1 * w2_ref[...], axis=1, keepdims=True) + b2_ref[...]
        p = jax.nn.sigmoid(zs)
        o_ref[...] = jnp.broadcast_to(p, (p.shape[0], _D))

    return pl.pallas_call(
        kern,
        grid=(pl.cdiv(_NPAD, bm),),
        in_specs=[
            pl.BlockSpec((bm, _D), lambda i: (i, 0)),
            pl.BlockSpec((bm, _D), lambda i: (i, 0)),
            pl.BlockSpec((_D, _D), lambda i: (0, 0)),
            pl.BlockSpec((_D, _D), lambda i: (0, 0)),
            pl.BlockSpec((1, _D), lambda i: (0, 0)),
            pl.BlockSpec((1, _D), lambda i: (0, 0)),
            pl.BlockSpec((1, 1), lambda i: (0, 0)),
        ],
        out_specs=pl.BlockSpec((bm, _D), lambda i: (i, 0)),
        out_shape=jax.ShapeDtypeStruct((_NPAD, _D), _f32),
    )(xo, h2, w1x, w1h, b1.reshape(1, -1), w2row, b2.reshape(1, 1))


_MESH = plsc.VectorSubcoreMesh(core_axis_name="c", subcore_axis_name="s")


def _sc_loopattr(dst, efc, zeros):
    @functools.partial(
        pl.kernel,
        mesh=_MESH,
        compiler_params=pltpu.CompilerParams(needs_layout_passes=False, use_tc_tiling_on_sc=False),
        out_type=jax.ShapeDtypeStruct((_NC, _NPAD, 32), _f32),
        scratch_types=[
            pltpu.VMEM((_BLK,), _i32),
            pltpu.VMEM((_BLK, 32), _f32),
            pltpu.VMEM_SHARED((_NPAD, 32), _f32),
        ],
    )
    def kern(dst_hbm, efc_hbm, zeros_hbm, out_hbm, idx_v, row_v, acc_sp):
        c = lax.axis_index("c")
        s = lax.axis_index("s")
        w = s * _NC + c
        r0 = s * _ROWS_T
        pltpu.sync_copy(zeros_hbm.at[pl.ds(r0, _ROWS_T), :],
                        acc_sp.at[pl.ds(r0, _ROWS_T), :])
        plsc.subcore_barrier()

        def blk(b, carry):
            base = (w * _LBLK + b) * _BLK
            pltpu.sync_copy(dst_hbm.at[pl.ds(base, _BLK)], idx_v)
            pltpu.sync_copy(efc_hbm.at[pl.ds(base, _BLK), :], row_v)
            pltpu.sync_copy(row_v, acc_sp.at[idx_v], add=True)
            return carry

        lax.fori_loop(0, _LBLK, blk, 0)
        plsc.subcore_barrier()
        pltpu.sync_copy(acc_sp.at[pl.ds(r0, _ROWS_T), :],
                        out_hbm.at[c, pl.ds(r0, _ROWS_T), :])

    return kern(dst, efc, zeros)


def _sc_pass1(src, dst, em, xl, xr, att, zeros, z16):
    @functools.partial(
        pl.kernel,
        mesh=_MESH,
        compiler_params=pltpu.CompilerParams(needs_layout_passes=False, use_tc_tiling_on_sc=False),
        out_type=[
            jax.ShapeDtypeStruct((_ESPAD, _H), _f32),
            jax.ShapeDtypeStruct((_NC, _NPAD, _D), _f32),
            jax.ShapeDtypeStruct((_NC, _NPAD, _H), _f32),
        ],
        scratch_types=[
            pltpu.VMEM((_BLK,), _i32),
            pltpu.VMEM((_BLK,), _i32),
            pltpu.VMEM((_BLK, _D), _f32),
            pltpu.VMEM((_BLK, _D), _f32),
            pltpu.VMEM((_BLK, _D), _f32),
            pltpu.VMEM((_BLK, _D), _f32),
            pltpu.VMEM((_BLK, _H), _f32),
            pltpu.VMEM((_D,), _f32),
            pltpu.VMEM_SHARED((_NPAD, _D), _f32),
            pltpu.VMEM_SHARED((_NPAD, _H), _f32),
            pltpu.SemaphoreType.DMA,
            pltpu.SemaphoreType.DMA,
        ],
    )
    def kern(src_hbm, dst_hbm, em_hbm, xl_hbm, xr_hbm, att_hbm, zeros_hbm,
             z16_hbm, ex_hbm, u_hbm, den_hbm,
             idx_s, idx_d, xlr, xrr, emr, wbuf, exbuf, attv, u_sp, den_sp,
             sem1, sem2):
        c = lax.axis_index("c")
        s = lax.axis_index("s")
        w = s * _NC + c
        r0 = s * _ROWS_T
        pltpu.sync_copy(zeros_hbm.at[pl.ds(r0, _ROWS_T), :],
                        u_sp.at[pl.ds(r0, _ROWS_T), :])
        pltpu.sync_copy(z16_hbm.at[pl.ds(r0, _ROWS_T), :],
                        den_sp.at[pl.ds(r0, _ROWS_T), :])
        pltpu.sync_copy(att_hbm, attv)
        plsc.subcore_barrier()

        ei = lax.iota(_i32, _BLK)

        def blk(b, carry):
            base = (w * _EBLK + b) * _BLK
            pltpu.sync_copy(src_hbm.at[pl.ds(base, _BLK)], idx_s)
            pltpu.sync_copy(dst_hbm.at[pl.ds(base, _BLK)], idx_d)
            cp1 = pltpu.async_copy(xl_hbm.at[idx_s], xlr, sem1)
            cp2 = pltpu.async_copy(xr_hbm.at[idx_d], xrr, sem2)
            pltpu.sync_copy(em_hbm.at[pl.ds(base, _BLK), :], emr)
            cp1.wait()
            cp2.wait()
            for h in range(_H):
                acc = jnp.zeros((_BLK,), _f32)
                xlc = []
                for cc in range(_C):
                    i = h * _C + cc
                    coli = jnp.full((_BLK,), i, _i32)
                    va = plsc.load_gather(xlr, [ei, coli])
                    vb = plsc.load_gather(xrr, [ei, coli])
                    ve = plsc.load_gather(emr, [ei, coli])
                    t = va + vb + ve
                    t = jnp.maximum(t, 0.2 * t)
                    vat = plsc.load_gather(attv, [coli])
                    acc = acc + t * vat
                    xlc.append(va)
                exh = jnp.exp(acc)
                plsc.store_scatter(exbuf, [ei, jnp.full((_BLK,), h, _i32)], exh)
                for cc in range(_C):
                    i = h * _C + cc
                    coli = jnp.full((_BLK,), i, _i32)
                    plsc.store_scatter(wbuf, [ei, coli], xlc[cc] * exh)
            pltpu.sync_copy(exbuf, ex_hbm.at[pl.ds(base, _BLK), :])
            pltpu.sync_copy(wbuf, u_sp.at[idx_d], add=True)
            pltpu.sync_copy(exbuf, den_sp.at[idx_d], add=True)
            return carry

        lax.fori_loop(0, _EBLK, blk, 0)
        plsc.subcore_barrier()
        pltpu.sync_copy(u_sp.at[pl.ds(r0, _ROWS_T), :],
                        u_hbm.at[c, pl.ds(r0, _ROWS_T), :])
        pltpu.sync_copy(den_sp.at[pl.ds(r0, _ROWS_T), :],
                        den_hbm.at[c, pl.ds(r0, _ROWS_T), :])

    return kern(src, dst, em, xl, xr, att, zeros, z16)


def _sc_pass2(dst, exb, dent):
    @functools.partial(
        pl.kernel,
        mesh=_MESH,
        compiler_params=pltpu.CompilerParams(needs_layout_passes=False, use_tc_tiling_on_sc=False),
        out_type=jax.ShapeDtypeStruct((_ESPAD, _H), _f32),
        scratch_types=[
            pltpu.VMEM((_BLK,), _i32),
            pltpu.VMEM((_BLK, _H), _f32),
            pltpu.VMEM((_BLK, _H), _f32),
            pltpu.VMEM((_BLK, _H), _f32),
            pltpu.SemaphoreType.DMA,
        ],
    )
    def kern(dst_hbm, ex_hbm, den_hbm, al_hbm, idx_d, exr, denr, abuf, sem):
        c = lax.axis_index("c")
        s = lax.axis_index("s")
        w = s * _NC + c

        def blk(b, carry):
            base = (w * _EBLK + b) * _BLK
            pltpu.sync_copy(dst_hbm.at[pl.ds(base, _BLK)], idx_d)
            cp = pltpu.async_copy(den_hbm.at[idx_d], denr, sem)
            pltpu.sync_copy(ex_hbm.at[pl.ds(base, _BLK), :], exr)
            cp.wait()
            for i in range(_BLK):
                abuf[i, :] = exr[i, :] / (denr[i, :] + 1e-16)
            pltpu.sync_copy(abuf, al_hbm.at[pl.ds(base, _BLK), :])
            return carry

        lax.fori_loop(0, _EBLK, blk, 0)

    return kern(dst, exb, dent)


def kernel(x, edge_index, edge_attr, batch, node_W, node_b, edge_W, edge_b,
           g1_ll_W, g1_ll_b, g1_lr_W, g1_lr_b, g1_le_W, g1_att, g1_bias,
           g2_ll_W, g2_ll_b, g2_lr_W, g2_lr_b, g2_le_W, g2_att, g2_bias,
           mlp_W1, mlp_b1, mlp_W2, mlp_b2):
    src = edge_index[0].astype(_i32)
    dst = edge_index[1].astype(_i32)
    loop = jnp.arange(_N, dtype=_i32)
    padi = jnp.full((_ESPAD - _ES,), 0, _i32)
    padd = jnp.full((_ESPAD - _ES,), _N, _i32)
    src_sl = jnp.concatenate([src, loop, padi])
    dst_sl = jnp.concatenate([dst, loop, padd])

    zeros = jnp.zeros((_NPAD, _D), _f32)
    z32 = jnp.zeros((_NPAD, 32), _f32)
    z16 = jnp.zeros((_NPAD, _H), _f32)
    rep = jnp.kron(jnp.eye(_H, dtype=_f32), jnp.ones((1, _C), _f32))

    x_p = jnp.pad(x, ((0, _NPAD - _N), (0, 0)))
    h = _mm(x_p, node_W.T, node_b)

    ebd = jnp.kron(jnp.eye(8, dtype=_f32), edge_W.T)
    ebias = jnp.tile(edge_b, 8)
    ef = _mm(edge_attr.reshape(_E // 8, 8 * _DE), ebd, ebias)
    ef = ef.reshape(_E, _DE)

    efc = jnp.concatenate([ef, jnp.ones((_E, 16), _f32)], axis=1)
    lap = _sc_loopattr(dst, efc, z32)
    loop_attr = _loopattr_finalize(lap[0], lap[1])[: _N]

    ef_sl = jnp.concatenate(
        [ef, loop_attr, jnp.zeros((_ESPAD - _ES, _DE), _f32)], axis=0)

    def gat_layer(h_in, ll_W, ll_b, lr_W, lr_b, le_W, att, bias):
        xl = _mm(h_in, ll_W.T, ll_b)
        xr = _mm(h_in, lr_W.T, lr_b)
        lbd = jnp.kron(jnp.eye(8, dtype=_f32), le_W.T)
        em = _mm(ef_sl.reshape(_ESPAD // 8, 8 * _DE), lbd,
                 jnp.zeros((8 * _D,), _f32), bm=2064)
        em = em.reshape(_ESPAD, _D)
        exb, u_p, den_p = _sc_pass1(src_sl, dst_sl, em, xl, xr,
                                    att.reshape(-1), zeros, z16)
        hn, dent = _combine(u_p[0], u_p[1], den_p[0], den_p[1], rep, bias)
        alpha = _sc_pass2(dst_sl, exb, dent)
        return hn, alpha[: _ES]

    h1, a1 = gat_layer(h, g1_ll_W, g1_ll_b, g1_lr_W, g1_lr_b,
                       g1_le_W, g1_att, g1_bias)
    h2, a2 = gat_layer(h1, g2_ll_W, g2_ll_b, g2_lr_W, g2_lr_b,
                       g2_le_W, g2_att, g2_bias)

    probs = _mlp(h, h2, mlp_W1.T[: _D], mlp_W1.T[_D:], mlp_b1,
                 mlp_W2, mlp_b2)[: _N, 0]
    return probs, a1, a2


# speedup vs baseline: 10.8825x; 1.3313x over previous
import functools

import jax
import jax.numpy as jnp
from jax import lax
from jax.experimental import pallas as pl
from jax.experimental.pallas import tpu as pltpu
from jax.experimental.pallas import tpu_sc as plsc

_N = 10000
_E = 320000
_H = 16
_C = 8
_D = 128
_DE = 16

_NC = 2
_NS = 16
_NW = _NC * _NS
_BLK = 16

_NPAD = 10112
_ROWS_T = _NPAD // _NS
_ES = _E + _N
_GB = 64
_SUB = _GB // _BLK
_GB2 = 128
_ESPAD = 331776
_EBLK = _ESPAD // (_NW * _GB)
_EBLK2 = _ESPAD // (_NW * _GB2)
_EPADL = 323584
_LBLK = _EPADL // (_NW * _GB2)

_f32 = jnp.float32
_i32 = jnp.int32


def _mm(a, bT, bias, act=None, bm=1024):
    M, K = a.shape
    _, Nn = bT.shape

    def kern(a_ref, b_ref, bias_ref, o_ref):
        r = jnp.dot(a_ref[...], b_ref[...], preferred_element_type=_f32)
        r = r + bias_ref[...]
        if act == "relu":
            r = jnp.maximum(r, 0.0)
        o_ref[...] = r

    return pl.pallas_call(
        kern,
        grid=(pl.cdiv(M, bm),),
        in_specs=[
            pl.BlockSpec((bm, K), lambda i: (i, 0)),
            pl.BlockSpec((K, Nn), lambda i: (0, 0)),
            pl.BlockSpec((1, Nn), lambda i: (0, 0)),
        ],
        out_specs=pl.BlockSpec((bm, Nn), lambda i: (i, 0)),
        out_shape=jax.ShapeDtypeStruct((M, Nn), _f32),
    )(a, bT, bias.reshape(1, -1))


def _loopattr_finalize(sc0, sc1, bm=1024):
    def kern(a_ref, b_ref, o_ref):
        t = a_ref[...] + b_ref[...]
        s = t[:, 0:16]
        cnt = t[:, 16:32]
        o_ref[...] = s / jnp.maximum(cnt, 1.0)

    return pl.pallas_call(
        kern,
        grid=(pl.cdiv(_NPAD, bm),),
        in_specs=[
            pl.BlockSpec((bm, 32), lambda i: (i, 0)),
            pl.BlockSpec((bm, 32), lambda i: (i, 0)),
        ],
        out_specs=pl.BlockSpec((bm, 16), lambda i: (i, 0)),
        out_shape=jax.ShapeDtypeStruct((_NPAD, 16), _f32),
    )(sc0, sc1)


def _combine(u0, u1, d0, d1, rep, bias, bm=1024):
    def kern(u0_ref, u1_ref, d0_ref, d1_ref, r_ref, b_ref, hn_ref, dt_ref):
        dent = d0_ref[...] + d1_ref[...]
        den128 = jnp.dot(dent, r_ref[...], preferred_element_type=_f32)
        out = (u0_ref[...] + u1_ref[...]) / (den128 + 1e-16)
        hn_ref[...] = jnp.maximum(out + b_ref[...], 0.0)
        dt_ref[...] = dent

    return pl.pallas_call(
        kern,
        grid=(pl.cdiv(_NPAD, bm),),
        in_specs=[
            pl.BlockSpec((bm, _D), lambda i: (i, 0)),
            pl.BlockSpec((bm, _D), lambda i: (i, 0)),
            pl.BlockSpec((bm, _H), lambda i: (i, 0)),
            pl.BlockSpec((bm, _H), lambda i: (i, 0)),
            pl.BlockSpec((_H, _D), lambda i: (0, 0)),
            pl.BlockSpec((1, _D), lambda i: (0, 0)),
        ],
        out_specs=[
            pl.BlockSpec((bm, _D), lambda i: (i, 0)),
            pl.BlockSpec((bm, _H), lambda i: (i, 0)),
        ],
        out_shape=[
            jax.ShapeDtypeStruct((_NPAD, _D), _f32),
            jax.ShapeDtypeStruct((_NPAD, _H), _f32),
        ],
    )(u0, u1, d0, d1, rep, bias.reshape(1, -1))


def _mlp(xo, h2, w1x, w1h, b1, w2row, b2, bm=1024):
    def kern(x_ref, h_ref, wx_ref, wh_ref, b1_ref, w2_ref, b2_ref, o_ref):
        z1 = jnp.dot(x_ref[...], wx_ref[...], preferred_element_type=_f32)
        z1 = z1 + jnp.dot(h_ref[...], wh_ref[...], preferred_element_type=_f32)
        z1 = jnp.maximum(z1 + b1_ref[...], 0.0)
        zs = jnp.sum(z1 * w2_ref[...], axis=1, keepdims=True) + b2_ref[...]
        p = jax.nn.sigmoid(zs)
        o_ref[...] = jnp.broadcast_to(p, (p.shape[0], _D))

    return pl.pallas_call(
        kern,
        grid=(pl.cdiv(_NPAD, bm),),
        in_specs=[
            pl.BlockSpec((bm, _D), lambda i: (i, 0)),
            pl.BlockSpec((bm, _D), lambda i: (i, 0)),
            pl.BlockSpec((_D, _D), lambda i: (0, 0)),
            pl.BlockSpec((_D, _D), lambda i: (0, 0)),
            pl.BlockSpec((1, _D), lambda i: (0, 0)),
            pl.BlockSpec((1, _D), lambda i: (0, 0)),
            pl.BlockSpec((1, 1), lambda i: (0, 0)),
        ],
        out_specs=pl.BlockSpec((bm, _D), lambda i: (i, 0)),
        out_shape=jax.ShapeDtypeStruct((_NPAD, _D), _f32),
    )(xo, h2, w1x, w1h, b1.reshape(1, -1), w2row, b2.reshape(1, 1))


_MESH = plsc.VectorSubcoreMesh(core_axis_name="c", subcore_axis_name="s")


def _sc_loopattr(dst, efc, zeros):
    @functools.partial(
        pl.kernel,
        mesh=_MESH,
        compiler_params=pltpu.CompilerParams(needs_layout_passes=False, use_tc_tiling_on_sc=False),
        out_type=jax.ShapeDtypeStruct((_NC, _NPAD, 32), _f32),
        scratch_types=[
            pltpu.VMEM((_GB2,), _i32),
            pltpu.VMEM((_GB2, 32), _f32),
            pltpu.VMEM_SHARED((_NPAD, 32), _f32),
        ],
    )
    def kern(dst_hbm, efc_hbm, zeros_hbm, out_hbm, idx_v, row_v, acc_sp):
        c = lax.axis_index("c")
        s = lax.axis_index("s")
        w = s * _NC + c
        r0 = s * _ROWS_T
        pltpu.sync_copy(zeros_hbm.at[pl.ds(r0, _ROWS_T), :],
                        acc_sp.at[pl.ds(r0, _ROWS_T), :])
        plsc.subcore_barrier()

        def blk(b, carry):
            base = (w * _LBLK + b) * _GB2
            pltpu.sync_copy(dst_hbm.at[pl.ds(base, _GB2)], idx_v)
            pltpu.sync_copy(efc_hbm.at[pl.ds(base, _GB2), :], row_v)
            pltpu.sync_copy(row_v, acc_sp.at[idx_v], add=True)
            return carry

        lax.fori_loop(0, _LBLK, blk, 0)
        plsc.subcore_barrier()
        pltpu.sync_copy(acc_sp.at[pl.ds(r0, _ROWS_T), :],
                        out_hbm.at[c, pl.ds(r0, _ROWS_T), :])

    return kern(dst, efc, zeros)


def _sc_pass1(src, dst, em, xl, xr, att, zeros, z16):
    @functools.partial(
        pl.kernel,
        mesh=_MESH,
        compiler_params=pltpu.CompilerParams(needs_layout_passes=False, use_tc_tiling_on_sc=False),
        out_type=[
            jax.ShapeDtypeStruct((_ESPAD, _H), _f32),
            jax.ShapeDtypeStruct((_NC, _NPAD, _D), _f32),
            jax.ShapeDtypeStruct((_NC, _NPAD, _H), _f32),
        ],
        scratch_types=[
            pltpu.VMEM((_GB,), _i32),
            pltpu.VMEM((_GB,), _i32),
            pltpu.VMEM((_GB, _D), _f32),
            pltpu.VMEM((_GB, _D), _f32),
            pltpu.VMEM((_GB, _D), _f32),
            pltpu.VMEM((_GB, _D), _f32),
            pltpu.VMEM((_GB, _H), _f32),
            pltpu.VMEM((_D,), _f32),
            pltpu.VMEM_SHARED((_NPAD, _D), _f32),
            pltpu.VMEM_SHARED((_NPAD, _H), _f32),
            pltpu.SemaphoreType.DMA,
            pltpu.SemaphoreType.DMA,
        ],
    )
    def kern(src_hbm, dst_hbm, em_hbm, xl_hbm, xr_hbm, att_hbm, zeros_hbm,
             z16_hbm, ex_hbm, u_hbm, den_hbm,
             idx_s, idx_d, xlr, xrr, emr, wbuf, exbuf, attv, u_sp, den_sp,
             sem1, sem2):
        c = lax.axis_index("c")
        s = lax.axis_index("s")
        w = s * _NC + c
        r0 = s * _ROWS_T
        pltpu.sync_copy(zeros_hbm.at[pl.ds(r0, _ROWS_T), :],
                        u_sp.at[pl.ds(r0, _ROWS_T), :])
        pltpu.sync_copy(z16_hbm.at[pl.ds(r0, _ROWS_T), :],
                        den_sp.at[pl.ds(r0, _ROWS_T), :])
        pltpu.sync_copy(att_hbm, attv)
        plsc.subcore_barrier()

        ei0 = lax.iota(_i32, _BLK)

        def sub(si, carry):
            ei = ei0 + si * _BLK
            for h in range(_H):
                acc = jnp.zeros((_BLK,), _f32)
                xlc = []
                for cc in range(_C):
                    i = h * _C + cc
                    coli = jnp.full((_BLK,), i, _i32)
                    va = plsc.load_gather(xlr, [ei, coli])
                    vb = plsc.load_gather(xrr, [ei, coli])
                    ve = plsc.load_gather(emr, [ei, coli])
                    t = va + vb + ve
                    t = jnp.maximum(t, 0.2 * t)
                    vat = plsc.load_gather(attv, [coli])
                    acc = acc + t * vat
                    xlc.append(va)
                exh = jnp.exp(acc)
                plsc.store_scatter(exbuf, [ei, jnp.full((_BLK,), h, _i32)], exh)
                for cc in range(_C):
                    i = h * _C + cc
                    coli = jnp.full((_BLK,), i, _i32)
                    plsc.store_scatter(wbuf, [ei, coli], xlc[cc] * exh)
            return carry

        def blk(b, carry):
            base = (w * _EBLK + b) * _GB
            pltpu.sync_copy(src_hbm.at[pl.ds(base, _GB)], idx_s)
            pltpu.sync_copy(dst_hbm.at[pl.ds(base, _GB)], idx_d)
            cp1 = pltpu.async_copy(xl_hbm.at[idx_s], xlr, sem1)
            cp2 = pltpu.async_copy(xr_hbm.at[idx_d], xrr, sem2)
            pltpu.sync_copy(em_hbm.at[pl.ds(base, _GB), :], emr)
            cp1.wait()
            cp2.wait()
            lax.fori_loop(0, _SUB, sub, 0)
            pltpu.sync_copy(exbuf, ex_hbm.at[pl.ds(base, _GB), :])
            pltpu.sync_copy(wbuf, u_sp.at[idx_d], add=True)
            pltpu.sync_copy(exbuf, den_sp.at[idx_d], add=True)
            return carry

        lax.fori_loop(0, _EBLK, blk, 0)
        plsc.subcore_barrier()
        pltpu.sync_copy(u_sp.at[pl.ds(r0, _ROWS_T), :],
                        u_hbm.at[c, pl.ds(r0, _ROWS_T), :])
        pltpu.sync_copy(den_sp.at[pl.ds(r0, _ROWS_T), :],
                        den_hbm.at[c, pl.ds(r0, _ROWS_T), :])

    return kern(src, dst, em, xl, xr, att, zeros, z16)


def _sc_pass2(dst, exb, dent):
    @functools.partial(
        pl.kernel,
        mesh=_MESH,
        compiler_params=pltpu.CompilerParams(needs_layout_passes=False, use_tc_tiling_on_sc=False),
        out_type=jax.ShapeDtypeStruct((_ESPAD, _H), _f32),
        scratch_types=[
            pltpu.VMEM((_GB2,), _i32),
            pltpu.VMEM((_GB2, _H), _f32),
            pltpu.VMEM((_GB2, _H), _f32),
            pltpu.VMEM((_GB2, _H), _f32),
            pltpu.SemaphoreType.DMA,
        ],
    )
    def kern(dst_hbm, ex_hbm, den_hbm, al_hbm, idx_d, exr, denr, abuf, sem):
        c = lax.axis_index("c")
        s = lax.axis_index("s")
        w = s * _NC + c

        def blk(b, carry):
            base = (w * _EBLK2 + b) * _GB2
            pltpu.sync_copy(dst_hbm.at[pl.ds(base, _GB2)], idx_d)
            cp = pltpu.async_copy(den_hbm.at[idx_d], denr, sem)
            pltpu.sync_copy(ex_hbm.at[pl.ds(base, _GB2), :], exr)
            cp.wait()
            for i in range(_GB2):
                abuf[i, :] = exr[i, :] / (denr[i, :] + 1e-16)
            pltpu.sync_copy(abuf, al_hbm.at[pl.ds(base, _GB2), :])
            return carry

        lax.fori_loop(0, _EBLK2, blk, 0)

    return kern(dst, exb, dent)


def kernel(x, edge_index, edge_attr, batch, node_W, node_b, edge_W, edge_b,
           g1_ll_W, g1_ll_b, g1_lr_W, g1_lr_b, g1_le_W, g1_att, g1_bias,
           g2_ll_W, g2_ll_b, g2_lr_W, g2_lr_b, g2_le_W, g2_att, g2_bias,
           mlp_W1, mlp_b1, mlp_W2, mlp_b2):
    src = edge_index[0].astype(_i32)
    dst = edge_index[1].astype(_i32)
    loop = jnp.arange(_N, dtype=_i32)
    padi = jnp.full((_ESPAD - _ES,), 0, _i32)
    padd = jnp.full((_ESPAD - _ES,), _N, _i32)
    src_sl = jnp.concatenate([src, loop, padi])
    dst_sl = jnp.concatenate([dst, loop, padd])

    zeros = jnp.zeros((_NPAD, _D), _f32)
    z32 = jnp.zeros((_NPAD, 32), _f32)
    z16 = jnp.zeros((_NPAD, _H), _f32)
    rep = jnp.kron(jnp.eye(_H, dtype=_f32), jnp.ones((1, _C), _f32))

    x_p = jnp.pad(x, ((0, _NPAD - _N), (0, 0)))
    h = _mm(x_p, node_W.T, node_b)

    ebd = jnp.kron(jnp.eye(8, dtype=_f32), edge_W.T)
    ebias = jnp.tile(edge_b, 8)
    ef = _mm(edge_attr.reshape(_E // 8, 8 * _DE), ebd, ebias)
    ef = ef.reshape(_E, _DE)

    efc = jnp.concatenate([ef, jnp.ones((_E, 16), _f32)], axis=1)
    efc = jnp.pad(efc, ((0, _EPADL - _E), (0, 0)))
    dst_l = jnp.concatenate([dst, jnp.full((_EPADL - _E,), _N, _i32)])
    lap = _sc_loopattr(dst_l, efc, z32)
    loop_attr = _loopattr_finalize(lap[0], lap[1])[: _N]

    ef_sl = jnp.concatenate(
        [ef, loop_attr, jnp.zeros((_ESPAD - _ES, _DE), _f32)], axis=0)

    def gat_layer(h_in, ll_W, ll_b, lr_W, lr_b, le_W, att, bias):
        xl = _mm(h_in, ll_W.T, ll_b)
        xr = _mm(h_in, lr_W.T, lr_b)
        lbd = jnp.kron(jnp.eye(8, dtype=_f32), le_W.T)
        em = _mm(ef_sl.reshape(_ESPAD // 8, 8 * _DE), lbd,
                 jnp.zeros((8 * _D,), _f32), bm=1728)
        em = em.reshape(_ESPAD, _D)
        exb, u_p, den_p = _sc_pass1(src_sl, dst_sl, em, xl, xr,
                                    att.reshape(-1), zeros, z16)
        hn, dent = _combine(u_p[0], u_p[1], den_p[0], den_p[1], rep, bias)
        alpha = _sc_pass2(dst_sl, exb, dent)
        return hn, alpha[: _ES]

    h1, a1 = gat_layer(h, g1_ll_W, g1_ll_b, g1_lr_W, g1_lr_b,
                       g1_le_W, g1_att, g1_bias)
    h2, a2 = gat_layer(h1, g2_ll_W, g2_ll_b, g2_lr_W, g2_lr_b,
                       g2_le_W, g2_att, g2_bias)

    probs = _mlp(h, h2, mlp_W1.T[: _D], mlp_W1.T[_D:], mlp_b1,
                 mlp_W2, mlp_b2)[: _N, 0]
    return probs, a1, a2
